# Initial kernel scaffold; baseline (speedup 1.0000x reference)
import jax
import jax.numpy as jnp
from jax.experimental import pallas as pl


def kernel(x, edge_index, batch_idx, W_emb, b_emb, W_g0, b_g0, W_g1, b_g1, W_g2, b_g2, W_ih, W_hh, b_ih, b_hh, W_in, b_in, W_out, b_out, W_c1, b_c1, W_c2, b_c2):
    raise NotImplementedError("write your pallas kernel here")


# speedup vs baseline: 12.2581x; 12.2581x over previous
import functools

import jax
import jax.numpy as jnp
from jax import lax
from jax.experimental import pallas as pl
from jax.experimental.pallas import tpu as pltpu
from jax.experimental.pallas import tpu_sc as plsc

N = 50000
E = 800000
F_IN = 64
H = 128
G = 128
NC = 2

NSC = 2
NSUB = 16
HC = H // 4

EB = 128
KB = 4
NGRP = 98
ET = EB * KB * NGRP
EP = ET * NSUB
ER = EP // EB

NP = ET
NPAD = NP
RT = NP // NSUB
ZR = 224

KBD = 7
NGD = 28

RB = 3584
NB = NPAD // RB

_PREC = lax.Precision.HIGHEST
_mesh = functools.partial(
    plsc.VectorSubcoreMesh, core_axis_name="c", subcore_axis_name="s",
    num_cores=NSC, num_subcores=NSUB)
_SC_PARAMS = pltpu.CompilerParams(use_tc_tiling_on_sc=False)


def _sc_deg_body(dst2, deg_out, acc, dstbuf, onesbuf, zbuf):
    cid = lax.axis_index("c")
    sid = lax.axis_index("s")
    zero16 = jnp.zeros((16,), jnp.float32)
    one16 = jnp.ones((16,), jnp.float32)

    def init(i, _):
        zbuf[pl.ds(i * 16, 16)] = zero16
        return 0
    lax.fori_loop(0, (NP // NSUB) // 16, init, 0)

    def initone(i, _):
        onesbuf[pl.ds(i * 16, 16)] = one16
        return 0
    lax.fori_loop(0, EB // 16, initone, 0)

    pltpu.sync_copy(zbuf, acc.at[pl.ds(sid * (NP // NSUB), NP // NSUB)])
    plsc.subcore_barrier()

    def grp(g, _):
        row0 = cid * (ER // NSC) + sid * (KBD * NGD) + g * KBD
        pltpu.sync_copy(dst2.at[pl.ds(row0, KBD)], dstbuf)
        for j in range(KBD):
            pltpu.sync_copy(onesbuf, acc.at[dstbuf.at[j]], add=True)
        return 0
    lax.fori_loop(0, NGD, grp, 0)
    plsc.subcore_barrier()

    pltpu.sync_copy(acc.at[pl.ds(sid * (NP // NSUB), NP // NSUB)],
                    deg_out.at[cid, pl.ds(sid * (NP // NSUB), NP // NSUB)])


_sc_deg = pl.kernel(
    _sc_deg_body,
    out_type=jax.ShapeDtypeStruct((NSC, NP), jnp.float32),
    mesh=_mesh(),
    compiler_params=_SC_PARAMS,
    scratch_types=[
        pltpu.VMEM_SHARED((NP,), jnp.float32),
        pltpu.VMEM((KBD, EB), jnp.int32),
        pltpu.VMEM((EB,), jnp.float32),
        pltpu.VMEM((NP // NSUB,), jnp.float32),
    ],
)


def _sc_scatter_body(y4, src2, dst2, a_out, acc, srcbuf, dstbuf, s4buf,
                     rowbuf, zbuf, gsem):
    cid = lax.axis_index("c")
    sid = lax.axis_index("s")
    zero16 = jnp.zeros((16,), jnp.float32)

    def initz(i, _):
        zbuf[i // 2, pl.ds((i % 2) * 16, 16)] = zero16
        return 0
    lax.fori_loop(0, ZR * 2, initz, 0)

    for cc in range(2):
        ck = cid * 2 + cc

        def zc(j, _):
            pltpu.sync_copy(
                zbuf, acc.at[pl.ds(sid * (NP // NSUB) + j * ZR, ZR)])
            return 0
        lax.fori_loop(0, (NP // NSUB) // ZR, zc, 0)
        plsc.subcore_barrier()

        def grp(g, _):
            row0 = sid * (KB * NGRP) + g * KB
            pltpu.sync_copy(src2.at[pl.ds(row0, KB)], srcbuf)
            pltpu.sync_copy(dst2.at[pl.ds(row0, KB)], dstbuf)

            def mk(i, _):
                j = i // (EB // 16)
                k = i % (EB // 16)
                v = srcbuf[j, pl.ds(k * 16, 16)]
                s4buf[j, pl.ds(k * 16, 16)] = v * 4 + ck
                return 0
            lax.fori_loop(0, KB * (EB // 16), mk, 0)

            descs = [
                pltpu.async_copy(
                    y4.at[s4buf.at[j]],
                    rowbuf.at[pl.ds(j * EB, EB)], gsem)
                for j in range(KB)
            ]
            for d in descs:
                d.wait()
            for j in range(KB):
                pltpu.sync_copy(rowbuf.at[pl.ds(j * EB, EB)],
                                acc.at[dstbuf.at[j]], add=True)
            return 0
        lax.fori_loop(0, NGRP, grp, 0)
        plsc.subcore_barrier()

        pltpu.sync_copy(acc.at[pl.ds(sid * RT, RT)],
                        a_out.at[pl.ds(sid * RT, RT), pl.ds(ck * HC, HC)])
        plsc.subcore_barrier()


_sc_scatter = pl.kernel(
    _sc_scatter_body,
    out_type=jax.ShapeDtypeStruct((NPAD, H), jnp.float32),
    mesh=_mesh(),
    compiler_params=_SC_PARAMS,
    scratch_types=[
        pltpu.VMEM_SHARED((NP, HC), jnp.float32),
        pltpu.VMEM((KB, EB), jnp.int32),
        pltpu.VMEM((KB, EB), jnp.int32),
        pltpu.VMEM((KB, EB), jnp.int32),
        pltpu.VMEM((KB * EB, HC), jnp.float32),
        pltpu.VMEM((ZR, HC), jnp.float32),
        pltpu.SemaphoreType.DMA,
    ],
)


def _tc_pre_body(x_ref, deg_ref, wemb_ref, bemb_ref, wg0_ref,
                 y0_ref, dinv_ref):
    deg = deg_ref[0, :] + deg_ref[1, :] + 1.0
    dinv = lax.rsqrt(deg)[:, None]
    h = jnp.dot(x_ref[...], wemb_ref[...].T, precision=_PREC) + bemb_ref[...]
    y0_ref[...] = jnp.dot(h, wg0_ref[...].T, precision=_PREC) * dinv
    dinv_ref[...] = dinv


def _tc_pre(x, deg2, w_emb, b_emb, w_g0):
    return pl.pallas_call(
        _tc_pre_body,
        grid=(NB,),
        in_specs=[
            pl.BlockSpec((RB, F_IN), lambda i: (i, 0)),
            pl.BlockSpec((NSC, RB), lambda i: (0, i)),
            pl.BlockSpec((H, F_IN), lambda i: (0, 0)),
            pl.BlockSpec((1, H), lambda i: (0, 0)),
            pl.BlockSpec((H, H), lambda i: (0, 0)),
        ],
        out_specs=[
            pl.BlockSpec((RB, H), lambda i: (i, 0)),
            pl.BlockSpec((RB, 1), lambda i: (i, 0)),
        ],
        out_shape=[
            jax.ShapeDtypeStruct((NPAD, H), jnp.float32),
            jax.ShapeDtypeStruct((NPAD, 1), jnp.float32),
        ],
    )(x, deg2, w_emb, b_emb.reshape(1, H), w_g0)


def _tc_mid_body(a_ref, y_ref, dinv_ref, b_ref, w_ref, yout_ref):
    dinv = dinv_ref[...]
    h = jnp.maximum(dinv * (a_ref[...] + y_ref[...]) + b_ref[...], 0.0)
    yout_ref[...] = jnp.dot(h, w_ref[...].T, precision=_PREC) * dinv


def _tc_mid(a, y, dinv, b, w):
    return pl.pallas_call(
        _tc_mid_body,
        grid=(NB,),
        in_specs=[
            pl.BlockSpec((RB, H), lambda i: (i, 0)),
            pl.BlockSpec((RB, H), lambda i: (i, 0)),
            pl.BlockSpec((RB, 1), lambda i: (i, 0)),
            pl.BlockSpec((1, H), lambda i: (0, 0)),
            pl.BlockSpec((H, H), lambda i: (0, 0)),
        ],
        out_specs=pl.BlockSpec((RB, H), lambda i: (i, 0)),
        out_shape=jax.ShapeDtypeStruct((NPAD, H), jnp.float32),
    )(a, y, dinv, b.reshape(1, H), w)


def _tc_final_body(a_ref, y_ref, dinv_ref, bg2_ref, bidx_ref,
                   wih_ref, bih_ref, bhh_ref, win_ref, bin_ref,
                   wout_ref, bout_ref, wc1_ref, bc1_ref, wc2_ref, bc2_ref,
                   out_ref, sums, cnt):
    i = pl.program_id(0)

    @pl.when(i == 0)
    def _():
        sums[...] = jnp.zeros_like(sums)
        cnt[...] = jnp.zeros_like(cnt)

    dinv = dinv_ref[...]
    h3 = jnp.maximum(dinv * (a_ref[...] + y_ref[...]) + bg2_ref[...], 0.0)
    b = bidx_ref[0, 0, :]
    onehot = (lax.broadcasted_iota(jnp.int32, (G, RB), 0)
              == b[None, :]).astype(jnp.float32)
    sums[...] += jnp.dot(onehot, h3, precision=_PREC)
    cnt[...] += jnp.sum(onehot, axis=1, keepdims=True)

    @pl.when(i == NB - 1)
    def _():
        ge = sums[...] / jnp.maximum(cnt[...], 1.0)
        gates = (jnp.dot(ge, wih_ref[...].T, precision=_PREC)
                 + bih_ref[...] + bhh_ref[...])
        i_g = gates[:, :H]
        g_g = gates[:, 2 * H:3 * H]
        o_g = gates[:, 3 * H:]
        c = jax.nn.sigmoid(i_g) * jnp.tanh(g_g)
        hl = jax.nn.sigmoid(o_g) * jnp.tanh(c)
        v = (jnp.dot(hl, win_ref[...][2 * H:3 * H, :].T, precision=_PREC)
             + bin_ref[...][:, 2 * H:3 * H])
        o = jnp.dot(v, wout_ref[...].T, precision=_PREC) + bout_ref[...]
        h1 = jnp.maximum(
            jnp.dot(o, wc1_ref[...].T, precision=_PREC) + bc1_ref[...], 0.0)
        out_ref[...] = jnp.dot(h1, wc2_ref[...].T, precision=_PREC) + bc2_ref[...]


def _tc_final(a, y, dinv, bg2, bidx3, w_ih, b_ih, b_hh, w_in, b_in,
              w_out, b_out, w_c1, b_c1, w_c2, b_c2):
    full = lambda r, c: pl.BlockSpec((r, c), lambda i: (0, 0))
    return pl.pallas_call(
        _tc_final_body,
        grid=(NB,),
        in_specs=[
            pl.BlockSpec((RB, H), lambda i: (i, 0)),
            pl.BlockSpec((RB, H), lambda i: (i, 0)),
            pl.BlockSpec((RB, 1), lambda i: (i, 0)),
            full(1, H),
            pl.BlockSpec((1, 1, RB), lambda i: (i, 0, 0)),
            full(4 * H, H), full(1, 4 * H), full(1, 4 * H),
            full(3 * H, H), full(1, 3 * H),
            full(H, H), full(1, H),
            full(H // 2, H), full(1, H // 2),
            full(NC, H // 2), full(1, NC),
        ],
        out_specs=pl.BlockSpec((G, NC), lambda i: (0, 0)),
        out_shape=jax.ShapeDtypeStruct((G, NC), jnp.float32),
        scratch_shapes=[
            pltpu.VMEM((G, H), jnp.float32),
            pltpu.VMEM((G, 1), jnp.float32),
        ],
    )(a, y, dinv, bg2.reshape(1, H), bidx3,
      w_ih, b_ih.reshape(1, 4 * H), b_hh.reshape(1, 4 * H),
      w_in, b_in.reshape(1, 3 * H),
      w_out, b_out.reshape(1, H),
      w_c1, b_c1.reshape(1, H // 2),
      w_c2, b_c2.reshape(1, NC))


def kernel(x, edge_index, batch_idx, W_emb, b_emb, W_g0, b_g0, W_g1, b_g1,
           W_g2, b_g2, W_ih, W_hh, b_ih, b_hh, W_in, b_in, W_out, b_out,
           W_c1, b_c1, W_c2, b_c2):
    src = edge_index[0]
    dst = edge_index[1]
    pad = EP - E
    srcp = jnp.concatenate([src, jnp.zeros((pad,), src.dtype)])
    dstp = jnp.concatenate([dst, jnp.full((pad,), N, dst.dtype)])
    src2 = srcp.reshape(ER, EB)
    dst2 = dstp.reshape(ER, EB)
    xp = jnp.concatenate([x, jnp.zeros((NPAD - N, F_IN), x.dtype)])
    bidxp = jnp.concatenate(
        [batch_idx, jnp.full((NPAD - N,), G, batch_idx.dtype)])
    bidx3 = bidxp.reshape(NB, 1, RB)

    deg2 = _sc_deg(dst2)
    y0, dinv = _tc_pre(xp, deg2, W_emb, b_emb, W_g0)
    a0 = _sc_scatter(y0.reshape(4 * NPAD, HC), src2, dst2)
    y1 = _tc_mid(a0, y0, dinv, b_g0, W_g1)
    a1 = _sc_scatter(y1.reshape(4 * NPAD, HC), src2, dst2)
    y2 = _tc_mid(a1, y1, dinv, b_g1, W_g2)
    a2 = _sc_scatter(y2.reshape(4 * NPAD, HC), src2, dst2)
    return _tc_final(a2, y2, dinv, b_g2, bidx3, W_ih, b_ih, b_hh,
                     W_in, b_in, W_out, b_out, W_c1, b_c1, W_c2, b_c2)


# speedup vs baseline: 13.8453x; 1.1295x over previous
import functools

import jax
import jax.numpy as jnp
from jax import lax
from jax.experimental import pallas as pl
from jax.experimental.pallas import tpu as pltpu
from jax.experimental.pallas import tpu_sc as plsc

N = 50000
E = 800000
F_IN = 64
H = 128
G = 128
NC = 2

NSC = 2
NSUB = 16
HC = H // 4

EB = 128
KB = 2
NGRP = 196
ET = EB * KB * NGRP
EP = ET * NSUB
ER = EP // EB

NP = ET
NPAD = NP
RT = NP // NSUB
ZR = 112

KBD = 7
NGD = 28

RB = 3584
NB = NPAD // RB

_PREC = lax.Precision.HIGHEST
_mesh = functools.partial(
    plsc.VectorSubcoreMesh, core_axis_name="c", subcore_axis_name="s",
    num_cores=NSC, num_subcores=NSUB)
_SC_PARAMS = pltpu.CompilerParams(use_tc_tiling_on_sc=False)


def _sc_deg_body(dst2, deg_out, acc, dstbuf, onesbuf, zbuf):
    cid = lax.axis_index("c")
    sid = lax.axis_index("s")
    zero16 = jnp.zeros((16,), jnp.float32)
    one16 = jnp.ones((16,), jnp.float32)

    def init(i, _):
        zbuf[pl.ds(i * 16, 16)] = zero16
        return 0
    lax.fori_loop(0, (NP // NSUB) // 16, init, 0)

    def initone(i, _):
        onesbuf[pl.ds(i * 16, 16)] = one16
        return 0
    lax.fori_loop(0, EB // 16, initone, 0)

    pltpu.sync_copy(zbuf, acc.at[pl.ds(sid * (NP // NSUB), NP // NSUB)])
    plsc.subcore_barrier()

    def grp(g, _):
        row0 = cid * (ER // NSC) + sid * (KBD * NGD) + g * KBD
        pltpu.sync_copy(dst2.at[pl.ds(row0, KBD)], dstbuf)
        for j in range(KBD):
            pltpu.sync_copy(onesbuf, acc.at[dstbuf.at[j]], add=True)
        return 0
    lax.fori_loop(0, NGD, grp, 0)
    plsc.subcore_barrier()

    pltpu.sync_copy(acc.at[pl.ds(sid * (NP // NSUB), NP // NSUB)],
                    deg_out.at[cid, pl.ds(sid * (NP // NSUB), NP // NSUB)])


_sc_deg = pl.kernel(
    _sc_deg_body,
    out_type=jax.ShapeDtypeStruct((NSC, NP), jnp.float32),
    mesh=_mesh(),
    compiler_params=_SC_PARAMS,
    scratch_types=[
        pltpu.VMEM_SHARED((NP,), jnp.float32),
        pltpu.VMEM((KBD, EB), jnp.int32),
        pltpu.VMEM((EB,), jnp.float32),
        pltpu.VMEM((NP // NSUB,), jnp.float32),
    ],
)


def _sc_scatter_body(y4, src2, dst2, a_out, acc,
                     srcA, dstA, s4A, rowA, srcB, dstB, s4B, rowB,
                     zbuf, gsem, ssem):
    cid = lax.axis_index("c")
    sid = lax.axis_index("s")
    zero16 = jnp.zeros((16,), jnp.float32)

    def initz(i, _):
        zbuf[i // 2, pl.ds((i % 2) * 16, 16)] = zero16
        return 0
    lax.fori_loop(0, ZR * 2, initz, 0)

    for cc in range(2):
        ck = cid * 2 + cc

        def zc(j, _):
            pltpu.sync_copy(
                zbuf, acc.at[pl.ds(sid * (NP // NSUB) + j * ZR, ZR)])
            return 0
        lax.fori_loop(0, (NP // NSUB) // ZR, zc, 0)
        plsc.subcore_barrier()

        def stage_fire(g, sbuf, dbuf, s4b, rbuf):
            row0 = sid * (KB * NGRP) + g * KB
            pltpu.sync_copy(src2.at[pl.ds(row0, KB)], sbuf)
            pltpu.sync_copy(dst2.at[pl.ds(row0, KB)], dbuf)

            def mk(i, _):
                j = i // (EB // 16)
                k = i % (EB // 16)
                v = sbuf[j, pl.ds(k * 16, 16)]
                s4b[j, pl.ds(k * 16, 16)] = v * 4 + ck
                return 0
            lax.fori_loop(0, KB * (EB // 16), mk, 0)
            for j in range(KB):
                pltpu.async_copy(y4.at[s4b.at[j]],
                                 rbuf.at[pl.ds(j * EB, EB)], gsem)

        def wait_gathers(s4b, rbuf):
            for j in range(KB):
                pltpu.make_async_copy(y4.at[s4b.at[j]],
                                      rbuf.at[pl.ds(j * EB, EB)],
                                      gsem).wait()

        def fire_scatters(dbuf, rbuf):
            for j in range(KB):
                pltpu.async_copy(rbuf.at[pl.ds(j * EB, EB)],
                                 acc.at[dbuf.at[j]], ssem, add=True)

        def wait_scatters(dbuf, rbuf):
            for j in range(KB):
                pltpu.make_async_copy(rbuf.at[pl.ds(j * EB, EB)],
                                      acc.at[dbuf.at[j]], ssem).wait()

        def half(g, cur, nxt, first, last):
            (sc, dc, s4c, rc) = cur
            (sn, dn, s4n, rn) = nxt
            if not first:
                wait_scatters(dn, rn)
            if not last:
                stage_fire(g + 1, sn, dn, s4n, rn)
            wait_gathers(s4c, rc)
            fire_scatters(dc, rc)

        bufA = (srcA, dstA, s4A, rowA)
        bufB = (srcB, dstB, s4B, rowB)
        stage_fire(0, *bufA)

        def grp2(p, _):
            g0 = 2 * p

            @pl.when(p == 0)
            def _():
                half(g0, bufA, bufB, True, False)

            @pl.when(p > 0)
            def _():
                half(g0, bufA, bufB, False, False)

            @pl.when(p < NGRP // 2 - 1)
            def _():
                half(g0 + 1, bufB, bufA, False, False)

            @pl.when(p == NGRP // 2 - 1)
            def _():
                half(g0 + 1, bufB, bufA, False, True)
            return 0
        lax.fori_loop(0, NGRP // 2, grp2, 0)
        wait_scatters(dstB, rowB)
        plsc.subcore_barrier()

        pltpu.sync_copy(acc.at[pl.ds(sid * RT, RT)],
                        a_out.at[pl.ds(sid * RT, RT), pl.ds(ck * HC, HC)])
        plsc.subcore_barrier()


_sc_scatter = pl.kernel(
    _sc_scatter_body,
    out_type=jax.ShapeDtypeStruct((NPAD, H), jnp.float32),
    mesh=_mesh(),
    compiler_params=_SC_PARAMS,
    scratch_types=[
        pltpu.VMEM_SHARED((NP, HC), jnp.float32),
        pltpu.VMEM((KB, EB), jnp.int32),
        pltpu.VMEM((KB, EB), jnp.int32),
        pltpu.VMEM((KB, EB), jnp.int32),
        pltpu.VMEM((KB * EB, HC), jnp.float32),
        pltpu.VMEM((KB, EB), jnp.int32),
        pltpu.VMEM((KB, EB), jnp.int32),
        pltpu.VMEM((KB, EB), jnp.int32),
        pltpu.VMEM((KB * EB, HC), jnp.float32),
        pltpu.VMEM((ZR, HC), jnp.float32),
        pltpu.SemaphoreType.DMA,
        pltpu.SemaphoreType.DMA,
    ],
)


def _tc_pre_body(x_ref, deg_ref, wemb_ref, bemb_ref, wg0_ref,
                 y0_ref, dinv_ref):
    deg = deg_ref[0, :] + deg_ref[1, :] + 1.0
    dinv = lax.rsqrt(deg)[:, None]
    h = jnp.dot(x_ref[...], wemb_ref[...].T, precision=_PREC) + bemb_ref[...]
    y0_ref[...] = jnp.dot(h, wg0_ref[...].T, precision=_PREC) * dinv
    dinv_ref[...] = dinv


def _tc_pre(x, deg2, w_emb, b_emb, w_g0):
    return pl.pallas_call(
        _tc_pre_body,
        grid=(NB,),
        in_specs=[
            pl.BlockSpec((RB, F_IN), lambda i: (i, 0)),
            pl.BlockSpec((NSC, RB), lambda i: (0, i)),
            pl.BlockSpec((H, F_IN), lambda i: (0, 0)),
            pl.BlockSpec((1, H), lambda i: (0, 0)),
            pl.BlockSpec((H, H), lambda i: (0, 0)),
        ],
        out_specs=[
            pl.BlockSpec((RB, H), lambda i: (i, 0)),
            pl.BlockSpec((RB, 1), lambda i: (i, 0)),
        ],
        out_shape=[
            jax.ShapeDtypeStruct((NPAD, H), jnp.float32),
            jax.ShapeDtypeStruct((NPAD, 1), jnp.float32),
        ],
    )(x, deg2, w_emb, b_emb.reshape(1, H), w_g0)


def _tc_mid_body(a_ref, y_ref, dinv_ref, b_ref, w_ref, yout_ref):
    dinv = dinv_ref[...]
    h = jnp.maximum(dinv * (a_ref[...] + y_ref[...]) + b_ref[...], 0.0)
    yout_ref[...] = jnp.dot(h, w_ref[...].T, precision=_PREC) * dinv


def _tc_mid(a, y, dinv, b, w):
    return pl.pallas_call(
        _tc_mid_body,
        grid=(NB,),
        in_specs=[
            pl.BlockSpec((RB, H), lambda i: (i, 0)),
            pl.BlockSpec((RB, H), lambda i: (i, 0)),
            pl.BlockSpec((RB, 1), lambda i: (i, 0)),
            pl.BlockSpec((1, H), lambda i: (0, 0)),
            pl.BlockSpec((H, H), lambda i: (0, 0)),
        ],
        out_specs=pl.BlockSpec((RB, H), lambda i: (i, 0)),
        out_shape=jax.ShapeDtypeStruct((NPAD, H), jnp.float32),
    )(a, y, dinv, b.reshape(1, H), w)


def _tc_final_body(a_ref, y_ref, dinv_ref, bg2_ref, bidx_ref,
                   wih_ref, bih_ref, bhh_ref, win_ref, bin_ref,
                   wout_ref, bout_ref, wc1_ref, bc1_ref, wc2_ref, bc2_ref,
                   out_ref, sums, cnt):
    i = pl.program_id(0)

    @pl.when(i == 0)
    def _():
        sums[...] = jnp.zeros_like(sums)
        cnt[...] = jnp.zeros_like(cnt)

    dinv = dinv_ref[...]
    h3 = jnp.maximum(dinv * (a_ref[...] + y_ref[...]) + bg2_ref[...], 0.0)
    b = bidx_ref[0, 0, :]
    onehot = (lax.broadcasted_iota(jnp.int32, (G, RB), 0)
              == b[None, :]).astype(jnp.float32)
    sums[...] += jnp.dot(onehot, h3, precision=_PREC)
    cnt[...] += jnp.sum(onehot, axis=1, keepdims=True)

    @pl.when(i == NB - 1)
    def _():
        ge = sums[...] / jnp.maximum(cnt[...], 1.0)
        gates = (jnp.dot(ge, wih_ref[...].T, precision=_PREC)
                 + bih_ref[...] + bhh_ref[...])
        i_g = gates[:, :H]
        g_g = gates[:, 2 * H:3 * H]
        o_g = gates[:, 3 * H:]
        c = jax.nn.sigmoid(i_g) * jnp.tanh(g_g)
        hl = jax.nn.sigmoid(o_g) * jnp.tanh(c)
        v = (jnp.dot(hl, win_ref[...][2 * H:3 * H, :].T, precision=_PREC)
             + bin_ref[...][:, 2 * H:3 * H])
        o = jnp.dot(v, wout_ref[...].T, precision=_PREC) + bout_ref[...]
        h1 = jnp.maximum(
            jnp.dot(o, wc1_ref[...].T, precision=_PREC) + bc1_ref[...], 0.0)
        out_ref[...] = jnp.dot(h1, wc2_ref[...].T, precision=_PREC) + bc2_ref[...]


def _tc_final(a, y, dinv, bg2, bidx3, w_ih, b_ih, b_hh, w_in, b_in,
              w_out, b_out, w_c1, b_c1, w_c2, b_c2):
    full = lambda r, c: pl.BlockSpec((r, c), lambda i: (0, 0))
    return pl.pallas_call(
        _tc_final_body,
        grid=(NB,),
        in_specs=[
            pl.BlockSpec((RB, H), lambda i: (i, 0)),
            pl.BlockSpec((RB, H), lambda i: (i, 0)),
            pl.BlockSpec((RB, 1), lambda i: (i, 0)),
            full(1, H),
            pl.BlockSpec((1, 1, RB), lambda i: (i, 0, 0)),
            full(4 * H, H), full(1, 4 * H), full(1, 4 * H),
            full(3 * H, H), full(1, 3 * H),
            full(H, H), full(1, H),
            full(H // 2, H), full(1, H // 2),
            full(NC, H // 2), full(1, NC),
        ],
        out_specs=pl.BlockSpec((G, NC), lambda i: (0, 0)),
        out_shape=jax.ShapeDtypeStruct((G, NC), jnp.float32),
        scratch_shapes=[
            pltpu.VMEM((G, H), jnp.float32),
            pltpu.VMEM((G, 1), jnp.float32),
        ],
    )(a, y, dinv, bg2.reshape(1, H), bidx3,
      w_ih, b_ih.reshape(1, 4 * H), b_hh.reshape(1, 4 * H),
      w_in, b_in.reshape(1, 3 * H),
      w_out, b_out.reshape(1, H),
      w_c1, b_c1.reshape(1, H // 2),
      w_c2, b_c2.reshape(1, NC))


def kernel(x, edge_index, batch_idx, W_emb, b_emb, W_g0, b_g0, W_g1, b_g1,
           W_g2, b_g2, W_ih, W_hh, b_ih, b_hh, W_in, b_in, W_out, b_out,
           W_c1, b_c1, W_c2, b_c2):
    src = edge_index[0]
    dst = edge_index[1]
    pad = EP - E
    srcp = jnp.concatenate([src, jnp.zeros((pad,), src.dtype)])
    dstp = jnp.concatenate([dst, jnp.full((pad,), N, dst.dtype)])
    src2 = srcp.reshape(ER, EB)
    dst2 = dstp.reshape(ER, EB)
    xp = jnp.concatenate([x, jnp.zeros((NPAD - N, F_IN), x.dtype)])
    bidxp = jnp.concatenate(
        [batch_idx, jnp.full((NPAD - N,), G, batch_idx.dtype)])
    bidx3 = bidxp.reshape(NB, 1, RB)

    deg2 = _sc_deg(dst2)
    y0, dinv = _tc_pre(xp, deg2, W_emb, b_emb, W_g0)
    a0 = _sc_scatter(y0.reshape(4 * NPAD, HC), src2, dst2)
    y1 = _tc_mid(a0, y0, dinv, b_g0, W_g1)
    a1 = _sc_scatter(y1.reshape(4 * NPAD, HC), src2, dst2)
    y2 = _tc_mid(a1, y1, dinv, b_g1, W_g2)
    a2 = _sc_scatter(y2.reshape(4 * NPAD, HC), src2, dst2)
    return _tc_final(a2, y2, dinv, b_g2, bidx3, W_ih, b_ih, b_hh,
                     W_in, b_in, W_out, b_out, W_c1, b_c1, W_c2, b_c2)


# speedup vs baseline: 20.0167x; 1.4457x over previous
import functools

import jax
import jax.numpy as jnp
from jax import lax
from jax.experimental import pallas as pl
from jax.experimental.pallas import tpu as pltpu
from jax.experimental.pallas import tpu_sc as plsc

N = 50000
E = 800000
F_IN = 64
H = 128
G = 128
NC = 2

NSC = 2
NSUB = 16
HC = H // 4

EB = 128
KB = 2
NGRP = 196
LSEG = 7
NBODY = NGRP // (2 * LSEG)
ET = EB * KB * NGRP
EP = ET * NSUB
ER = EP // EB

NP = ET
NPAD = NP
RT = NP // NSUB
ZR = 112

KBD = 7
NGD = 28

RB = 3584
NB = NPAD // RB

_PREC = lax.Precision.HIGHEST
_mesh = functools.partial(
    plsc.VectorSubcoreMesh, core_axis_name="c", subcore_axis_name="s",
    num_cores=NSC, num_subcores=NSUB)
_SC_PARAMS = pltpu.CompilerParams(use_tc_tiling_on_sc=False)


def _sc_deg_body(dst2, deg_out, acc, dstbuf, onesbuf, zbuf):
    cid = lax.axis_index("c")
    sid = lax.axis_index("s")
    zero16 = jnp.zeros((16,), jnp.float32)
    one16 = jnp.ones((16,), jnp.float32)

    def init(i, _):
        zbuf[pl.ds(i * 16, 16)] = zero16
        return 0
    lax.fori_loop(0, (NP // NSUB) // 16, init, 0)

    def initone(i, _):
        onesbuf[pl.ds(i * 16, 16)] = one16
        return 0
    lax.fori_loop(0, EB // 16, initone, 0)

    pltpu.sync_copy(zbuf, acc.at[pl.ds(sid * (NP // NSUB), NP // NSUB)])
    plsc.subcore_barrier()

    def grp(g, _):
        row0 = cid * (ER // NSC) + sid * (KBD * NGD) + g * KBD
        pltpu.sync_copy(dst2.at[pl.ds(row0, KBD)], dstbuf)
        for j in range(KBD):
            pltpu.sync_copy(onesbuf, acc.at[dstbuf.at[j]], add=True)
        return 0
    lax.fori_loop(0, NGD, grp, 0)
    plsc.subcore_barrier()

    pltpu.sync_copy(acc.at[pl.ds(sid * (NP // NSUB), NP // NSUB)],
                    deg_out.at[cid, pl.ds(sid * (NP // NSUB), NP // NSUB)])


_sc_deg = pl.kernel(
    _sc_deg_body,
    out_type=jax.ShapeDtypeStruct((NSC, NP), jnp.float32),
    mesh=_mesh(),
    compiler_params=_SC_PARAMS,
    scratch_types=[
        pltpu.VMEM_SHARED((NP,), jnp.float32),
        pltpu.VMEM((KBD, EB), jnp.int32),
        pltpu.VMEM((EB,), jnp.float32),
        pltpu.VMEM((NP // NSUB,), jnp.float32),
    ],
)


def _sc_scatter_body(y4, src4, dst2, a_out, acc,
                     s4A, dA, s4B, dB, rowA, rowB, zbuf,
                     gsem, ssem, isem):
    cid = lax.axis_index("c")
    sid = lax.axis_index("s")
    zero16 = jnp.zeros((16,), jnp.float32)
    SR = LSEG * KB

    def initz(i, _):
        zbuf[i // 2, pl.ds((i % 2) * 16, 16)] = zero16
        return 0
    lax.fori_loop(0, ZR * 2, initz, 0)

    for cc in range(2):
        ck = cid * 2 + cc

        def zc(j, _):
            pltpu.sync_copy(
                zbuf, acc.at[pl.ds(sid * (NP // NSUB) + j * ZR, ZR)])
            return 0
        lax.fori_loop(0, (NP // NSUB) // ZR, zc, 0)
        plsc.subcore_barrier()

        def fire_seg(sg, s4seg, dseg):
            row0 = sid * (NGRP * KB) + sg * SR
            pltpu.async_copy(src4.at[ck, pl.ds(row0, SR)], s4seg, isem)
            pltpu.async_copy(dst2.at[pl.ds(row0, SR)], dseg, isem)

        def wait_seg(s4seg, dseg):
            pltpu.make_async_copy(src4.at[ck, pl.ds(0, SR)], s4seg,
                                  isem).wait()
            pltpu.make_async_copy(dst2.at[pl.ds(0, SR)], dseg, isem).wait()

        def fire_gathers(s4seg, lr, rbuf):
            for j in range(KB):
                pltpu.async_copy(y4.at[s4seg.at[lr + j]],
                                 rbuf.at[pl.ds(j * EB, EB)], gsem)

        def wait_gathers(s4seg, lr, rbuf):
            for j in range(KB):
                pltpu.make_async_copy(y4.at[s4seg.at[lr + j]],
                                      rbuf.at[pl.ds(j * EB, EB)],
                                      gsem).wait()

        def fire_scatters(dseg, lr, rbuf):
            for j in range(KB):
                pltpu.async_copy(rbuf.at[pl.ds(j * EB, EB)],
                                 acc.at[dseg.at[lr + j]], ssem, add=True)

        def wait_scatters(dseg, lr, rbuf):
            for j in range(KB):
                pltpu.make_async_copy(rbuf.at[pl.ds(j * EB, EB)],
                                      acc.at[dseg.at[lr + j]], ssem).wait()

        def locbufs(lg):
            seg = (s4A, dA) if lg < LSEG else (s4B, dB)
            row = (rowA, rowB)[lg % 2]
            return seg[0], seg[1], (lg % LSEG) * KB, row

        def half(lg, first, last):
            s4c, dc, lrc, rc = locbufs(lg)
            if not first:
                s4p, dp, lrp, rp = locbufs((lg - 1) % (2 * LSEG))
                wait_scatters(dp, lrp, rp)
            if not last:
                s4n, dn, lrn, rn = locbufs((lg + 1) % (2 * LSEG))
                fire_gathers(s4n, lrn, rn)
            wait_gathers(s4c, lrc, rc)
            fire_scatters(dc, lrc, rc)

        fire_seg(0, s4A, dA)
        wait_seg(s4A, dA)
        fire_gathers(s4A, 0, rowA)

        def body(p, _):
            @pl.when(p == 0)
            def _():
                half(0, True, False)

            @pl.when(p > 0)
            def _():
                half(0, False, False)

            fire_seg(2 * p + 1, s4B, dB)
            for lg in range(1, LSEG - 1):
                half(lg, False, False)
            wait_seg(s4B, dB)
            half(LSEG - 1, False, False)
            half(LSEG, False, False)

            @pl.when(p < NBODY - 1)
            def _():
                fire_seg(2 * p + 2, s4A, dA)
            for lg in range(LSEG + 1, 2 * LSEG - 1):
                half(lg, False, False)

            @pl.when(p < NBODY - 1)
            def _():
                wait_seg(s4A, dA)
                half(2 * LSEG - 1, False, False)

            @pl.when(p == NBODY - 1)
            def _():
                half(2 * LSEG - 1, False, True)
            return 0
        lax.fori_loop(0, NBODY, body, 0)
        wait_scatters(dB, (LSEG - 1) * KB, rowB)
        plsc.subcore_barrier()

        pltpu.sync_copy(acc.at[pl.ds(sid * RT, RT)],
                        a_out.at[pl.ds(sid * RT, RT), pl.ds(ck * HC, HC)])
        plsc.subcore_barrier()


_sc_scatter = pl.kernel(
    _sc_scatter_body,
    out_type=jax.ShapeDtypeStruct((NPAD, H), jnp.float32),
    mesh=_mesh(),
    compiler_params=_SC_PARAMS,
    scratch_types=[
        pltpu.VMEM_SHARED((NP, HC), jnp.float32),
        pltpu.VMEM((LSEG * KB, EB), jnp.int32),
        pltpu.VMEM((LSEG * KB, EB), jnp.int32),
        pltpu.VMEM((LSEG * KB, EB), jnp.int32),
        pltpu.VMEM((LSEG * KB, EB), jnp.int32),
        pltpu.VMEM((KB * EB, HC), jnp.float32),
        pltpu.VMEM((KB * EB, HC), jnp.float32),
        pltpu.VMEM((ZR, HC), jnp.float32),
        pltpu.SemaphoreType.DMA,
        pltpu.SemaphoreType.DMA,
        pltpu.SemaphoreType.DMA,
    ],
)


def _tc_src4_body(src_ref, out_ref):
    s4 = src_ref[...] * 4
    for c in range(4):
        out_ref[c, :, :] = s4 + c


def _tc_src4(src2):
    rbe = ER // 8
    return pl.pallas_call(
        _tc_src4_body,
        grid=(8,),
        in_specs=[pl.BlockSpec((rbe, EB), lambda i: (i, 0))],
        out_specs=pl.BlockSpec((4, rbe, EB), lambda i: (0, i, 0)),
        out_shape=jax.ShapeDtypeStruct((4, ER, EB), jnp.int32),
    )(src2)


def _tc_pre_body(x_ref, deg_ref, wemb_ref, bemb_ref, wg0_ref,
                 y0_ref, dinv_ref):
    deg = deg_ref[0, :] + deg_ref[1, :] + 1.0
    dinv = lax.rsqrt(deg)[:, None]
    h = jnp.dot(x_ref[...], wemb_ref[...].T, precision=_PREC) + bemb_ref[...]
    y0_ref[...] = jnp.dot(h, wg0_ref[...].T, precision=_PREC) * dinv
    dinv_ref[...] = dinv


def _tc_pre(x, deg2, w_emb, b_emb, w_g0):
    return pl.pallas_call(
        _tc_pre_body,
        grid=(NB,),
        in_specs=[
            pl.BlockSpec((RB, F_IN), lambda i: (i, 0)),
            pl.BlockSpec((NSC, RB), lambda i: (0, i)),
            pl.BlockSpec((H, F_IN), lambda i: (0, 0)),
            pl.BlockSpec((1, H), lambda i: (0, 0)),
            pl.BlockSpec((H, H), lambda i: (0, 0)),
        ],
        out_specs=[
            pl.BlockSpec((RB, H), lambda i: (i, 0)),
            pl.BlockSpec((RB, 1), lambda i: (i, 0)),
        ],
        out_shape=[
            jax.ShapeDtypeStruct((NPAD, H), jnp.float32),
            jax.ShapeDtypeStruct((NPAD, 1), jnp.float32),
        ],
    )(x, deg2, w_emb, b_emb.reshape(1, H), w_g0)


def _tc_mid_body(a_ref, y_ref, dinv_ref, b_ref, w_ref, yout_ref):
    dinv = dinv_ref[...]
    h = jnp.maximum(dinv * (a_ref[...] + y_ref[...]) + b_ref[...], 0.0)
    yout_ref[...] = jnp.dot(h, w_ref[...].T, precision=_PREC) * dinv


def _tc_mid(a, y, dinv, b, w):
    return pl.pallas_call(
        _tc_mid_body,
        grid=(NB,),
        in_specs=[
            pl.BlockSpec((RB, H), lambda i: (i, 0)),
            pl.BlockSpec((RB, H), lambda i: (i, 0)),
            pl.BlockSpec((RB, 1), lambda i: (i, 0)),
            pl.BlockSpec((1, H), lambda i: (0, 0)),
            pl.BlockSpec((H, H), lambda i: (0, 0)),
        ],
        out_specs=pl.BlockSpec((RB, H), lambda i: (i, 0)),
        out_shape=jax.ShapeDtypeStruct((NPAD, H), jnp.float32),
    )(a, y, dinv, b.reshape(1, H), w)


def _tc_final_body(a_ref, y_ref, dinv_ref, bg2_ref, bidx_ref,
                   wih_ref, bih_ref, bhh_ref, win_ref, bin_ref,
                   wout_ref, bout_ref, wc1_ref, bc1_ref, wc2_ref, bc2_ref,
                   out_ref, sums, cnt):
    i = pl.program_id(0)

    @pl.when(i == 0)
    def _():
        sums[...] = jnp.zeros_like(sums)
        cnt[...] = jnp.zeros_like(cnt)

    dinv = dinv_ref[...]
    h3 = jnp.maximum(dinv * (a_ref[...] + y_ref[...]) + bg2_ref[...], 0.0)
    b = bidx_ref[0, 0, :]
    onehot = (lax.broadcasted_iota(jnp.int32, (G, RB), 0)
              == b[None, :]).astype(jnp.float32)
    sums[...] += jnp.dot(onehot, h3, precision=_PREC)
    cnt[...] += jnp.sum(onehot, axis=1, keepdims=True)

    @pl.when(i == NB - 1)
    def _():
        ge = sums[...] / jnp.maximum(cnt[...], 1.0)
        gates = (jnp.dot(ge, wih_ref[...].T, precision=_PREC)
                 + bih_ref[...] + bhh_ref[...])
        i_g = gates[:, :H]
        g_g = gates[:, 2 * H:3 * H]
        o_g = gates[:, 3 * H:]
        c = jax.nn.sigmoid(i_g) * jnp.tanh(g_g)
        hl = jax.nn.sigmoid(o_g) * jnp.tanh(c)
        v = (jnp.dot(hl, win_ref[...][2 * H:3 * H, :].T, precision=_PREC)
             + bin_ref[...][:, 2 * H:3 * H])
        o = jnp.dot(v, wout_ref[...].T, precision=_PREC) + bout_ref[...]
        h1 = jnp.maximum(
            jnp.dot(o, wc1_ref[...].T, precision=_PREC) + bc1_ref[...], 0.0)
        out_ref[...] = jnp.dot(h1, wc2_ref[...].T, precision=_PREC) + bc2_ref[...]


def _tc_final(a, y, dinv, bg2, bidx3, w_ih, b_ih, b_hh, w_in, b_in,
              w_out, b_out, w_c1, b_c1, w_c2, b_c2):
    full = lambda r, c: pl.BlockSpec((r, c), lambda i: (0, 0))
    return pl.pallas_call(
        _tc_final_body,
        grid=(NB,),
        in_specs=[
            pl.BlockSpec((RB, H), lambda i: (i, 0)),
            pl.BlockSpec((RB, H), lambda i: (i, 0)),
            pl.BlockSpec((RB, 1), lambda i: (i, 0)),
            full(1, H),
            pl.BlockSpec((1, 1, RB), lambda i: (i, 0, 0)),
            full(4 * H, H), full(1, 4 * H), full(1, 4 * H),
            full(3 * H, H), full(1, 3 * H),
            full(H, H), full(1, H),
            full(H // 2, H), full(1, H // 2),
            full(NC, H // 2), full(1, NC),
        ],
        out_specs=pl.BlockSpec((G, NC), lambda i: (0, 0)),
        out_shape=jax.ShapeDtypeStruct((G, NC), jnp.float32),
        scratch_shapes=[
            pltpu.VMEM((G, H), jnp.float32),
            pltpu.VMEM((G, 1), jnp.float32),
        ],
    )(a, y, dinv, bg2.reshape(1, H), bidx3,
      w_ih, b_ih.reshape(1, 4 * H), b_hh.reshape(1, 4 * H),
      w_in, b_in.reshape(1, 3 * H),
      w_out, b_out.reshape(1, H),
      w_c1, b_c1.reshape(1, H // 2),
      w_c2, b_c2.reshape(1, NC))


def kernel(x, edge_index, batch_idx, W_emb, b_emb, W_g0, b_g0, W_g1, b_g1,
           W_g2, b_g2, W_ih, W_hh, b_ih, b_hh, W_in, b_in, W_out, b_out,
           W_c1, b_c1, W_c2, b_c2):
    src = edge_index[0]
    dst = edge_index[1]
    pad = EP - E
    srcp = jnp.concatenate([src, jnp.zeros((pad,), src.dtype)])
    dstp = jnp.concatenate([dst, jnp.full((pad,), N, dst.dtype)])
    src2 = srcp.reshape(ER, EB)
    dst2 = dstp.reshape(ER, EB)
    xp = jnp.concatenate([x, jnp.zeros((NPAD - N, F_IN), x.dtype)])
    bidxp = jnp.concatenate(
        [batch_idx, jnp.full((NPAD - N,), G, batch_idx.dtype)])
    bidx3 = bidxp.reshape(NB, 1, RB)

    deg2 = _sc_deg(dst2)
    src4 = _tc_src4(src2)
    y0, dinv = _tc_pre(xp, deg2, W_emb, b_emb, W_g0)
    a0 = _sc_scatter(y0.reshape(4 * NPAD, HC), src4, dst2)
    y1 = _tc_mid(a0, y0, dinv, b_g0, W_g1)
    a1 = _sc_scatter(y1.reshape(4 * NPAD, HC), src4, dst2)
    y2 = _tc_mid(a1, y1, dinv, b_g1, W_g2)
    a2 = _sc_scatter(y2.reshape(4 * NPAD, HC), src4, dst2)
    return _tc_final(a2, y2, dinv, b_g2, bidx3, W_ih, b_ih, b_hh,
                     W_in, b_in, W_out, b_out, W_c1, b_c1, W_c2, b_c2)


# speedup vs baseline: 20.8857x; 1.0434x over previous
import functools

import jax
import jax.numpy as jnp
from jax import lax
from jax.experimental import pallas as pl
from jax.experimental.pallas import tpu as pltpu
from jax.experimental.pallas import tpu_sc as plsc

N = 50000
E = 800000
F_IN = 64
H = 128
G = 128
NC = 2

NSC = 2
NSUB = 16
HC = H // 4

EB = 128
KB = 2
NGRP = 196
LSEG = 7
NBODY = NGRP // (2 * LSEG)
ET = EB * KB * NGRP
EP = ET * NSUB
ER = EP // EB

NP = ET
NPAD = NP
RT = NP // NSUB
ZR = 112

KBD = 7
NGD = 28

RB = 3584
NB = NPAD // RB

_PREC = lax.Precision.DEFAULT
_mesh = functools.partial(
    plsc.VectorSubcoreMesh, core_axis_name="c", subcore_axis_name="s",
    num_cores=NSC, num_subcores=NSUB)
_SC_PARAMS = pltpu.CompilerParams(use_tc_tiling_on_sc=False)


def _sc_deg_body(dst2, deg_out, acc, dstbuf, onesbuf, zbuf):
    cid = lax.axis_index("c")
    sid = lax.axis_index("s")
    zero16 = jnp.zeros((16,), jnp.float32)
    one16 = jnp.ones((16,), jnp.float32)

    def init(i, _):
        zbuf[pl.ds(i * 16, 16)] = zero16
        return 0
    lax.fori_loop(0, (NP // NSUB) // 16, init, 0)

    def initone(i, _):
        onesbuf[pl.ds(i * 16, 16)] = one16
        return 0
    lax.fori_loop(0, EB // 16, initone, 0)

    pltpu.sync_copy(zbuf, acc.at[pl.ds(sid * (NP // NSUB), NP // NSUB)])
    plsc.subcore_barrier()

    def grp(g, _):
        row0 = cid * (ER // NSC) + sid * (KBD * NGD) + g * KBD
        pltpu.sync_copy(dst2.at[pl.ds(row0, KBD)], dstbuf)
        for j in range(KBD):
            pltpu.sync_copy(onesbuf, acc.at[dstbuf.at[j]], add=True)
        return 0
    lax.fori_loop(0, NGD, grp, 0)
    plsc.subcore_barrier()

    pltpu.sync_copy(acc.at[pl.ds(sid * (NP // NSUB), NP // NSUB)],
                    deg_out.at[cid, pl.ds(sid * (NP // NSUB), NP // NSUB)])


_sc_deg = pl.kernel(
    _sc_deg_body,
    out_type=jax.ShapeDtypeStruct((NSC, NP), jnp.float32),
    mesh=_mesh(),
    compiler_params=_SC_PARAMS,
    scratch_types=[
        pltpu.VMEM_SHARED((NP,), jnp.float32),
        pltpu.VMEM((KBD, EB), jnp.int32),
        pltpu.VMEM((EB,), jnp.float32),
        pltpu.VMEM((NP // NSUB,), jnp.float32),
    ],
)


def _sc_scatter_body(y4, src4, dst2, a_out, acc,
                     s4A, dA, s4B, dB, rowA, rowB, zbuf,
                     gsem, ssem, isem):
    cid = lax.axis_index("c")
    sid = lax.axis_index("s")
    zero16 = jnp.zeros((16,), jnp.float32)
    SR = LSEG * KB

    def initz(i, _):
        zbuf[i // 2, pl.ds((i % 2) * 16, 16)] = zero16
        return 0
    lax.fori_loop(0, ZR * 2, initz, 0)

    for cc in range(2):
        ck = cid * 2 + cc

        def zc(j, _):
            pltpu.sync_copy(
                zbuf, acc.at[pl.ds(sid * (NP // NSUB) + j * ZR, ZR)])
            return 0
        lax.fori_loop(0, (NP // NSUB) // ZR, zc, 0)
        plsc.subcore_barrier()

        def fire_seg(sg, s4seg, dseg):
            row0 = sid * (NGRP * KB) + sg * SR
            pltpu.async_copy(src4.at[ck, pl.ds(row0, SR)], s4seg, isem)
            pltpu.async_copy(dst2.at[pl.ds(row0, SR)], dseg, isem)

        def wait_seg(s4seg, dseg):
            pltpu.make_async_copy(src4.at[ck, pl.ds(0, SR)], s4seg,
                                  isem).wait()
            pltpu.make_async_copy(dst2.at[pl.ds(0, SR)], dseg, isem).wait()

        def fire_gathers(s4seg, lr, rbuf):
            for j in range(KB):
                pltpu.async_copy(y4.at[s4seg.at[lr + j]],
                                 rbuf.at[pl.ds(j * EB, EB)], gsem)

        def wait_gathers(s4seg, lr, rbuf):
            for j in range(KB):
                pltpu.make_async_copy(y4.at[s4seg.at[lr + j]],
                                      rbuf.at[pl.ds(j * EB, EB)],
                                      gsem).wait()

        def fire_scatters(dseg, lr, rbuf):
            for j in range(KB):
                pltpu.async_copy(rbuf.at[pl.ds(j * EB, EB)],
                                 acc.at[dseg.at[lr + j]], ssem, add=True)

        def wait_scatters(dseg, lr, rbuf):
            for j in range(KB):
                pltpu.make_async_copy(rbuf.at[pl.ds(j * EB, EB)],
                                      acc.at[dseg.at[lr + j]], ssem).wait()

        def locbufs(lg):
            seg = (s4A, dA) if lg < LSEG else (s4B, dB)
            row = (rowA, rowB)[lg % 2]
            return seg[0], seg[1], (lg % LSEG) * KB, row

        def half(lg, first, last):
            s4c, dc, lrc, rc = locbufs(lg)
            if not first:
                s4p, dp, lrp, rp = locbufs((lg - 1) % (2 * LSEG))
                wait_scatters(dp, lrp, rp)
            if not last:
                s4n, dn, lrn, rn = locbufs((lg + 1) % (2 * LSEG))
                fire_gathers(s4n, lrn, rn)
            wait_gathers(s4c, lrc, rc)
            fire_scatters(dc, lrc, rc)

        fire_seg(0, s4A, dA)
        wait_seg(s4A, dA)
        fire_gathers(s4A, 0, rowA)

        def body(p, _):
            @pl.when(p == 0)
            def _():
                half(0, True, False)

            @pl.when(p > 0)
            def _():
                half(0, False, False)

            fire_seg(2 * p + 1, s4B, dB)
            for lg in range(1, LSEG - 1):
                half(lg, False, False)
            wait_seg(s4B, dB)
            half(LSEG - 1, False, False)
            half(LSEG, False, False)

            @pl.when(p < NBODY - 1)
            def _():
                fire_seg(2 * p + 2, s4A, dA)
            for lg in range(LSEG + 1, 2 * LSEG - 1):
                half(lg, False, False)

            @pl.when(p < NBODY - 1)
            def _():
                wait_seg(s4A, dA)
                half(2 * LSEG - 1, False, False)

            @pl.when(p == NBODY - 1)
            def _():
                half(2 * LSEG - 1, False, True)
            return 0
        lax.fori_loop(0, NBODY, body, 0)
        wait_scatters(dB, (LSEG - 1) * KB, rowB)
        plsc.subcore_barrier()

        pltpu.sync_copy(acc.at[pl.ds(sid * RT, RT)],
                        a_out.at[pl.ds(sid * RT, RT), pl.ds(ck * HC, HC)])
        plsc.subcore_barrier()


_sc_scatter = pl.kernel(
    _sc_scatter_body,
    out_type=jax.ShapeDtypeStruct((NPAD, H), jnp.float32),
    mesh=_mesh(),
    compiler_params=_SC_PARAMS,
    scratch_types=[
        pltpu.VMEM_SHARED((NP, HC), jnp.float32),
        pltpu.VMEM((LSEG * KB, EB), jnp.int32),
        pltpu.VMEM((LSEG * KB, EB), jnp.int32),
        pltpu.VMEM((LSEG * KB, EB), jnp.int32),
        pltpu.VMEM((LSEG * KB, EB), jnp.int32),
        pltpu.VMEM((KB * EB, HC), jnp.float32),
        pltpu.VMEM((KB * EB, HC), jnp.float32),
        pltpu.VMEM((ZR, HC), jnp.float32),
        pltpu.SemaphoreType.DMA,
        pltpu.SemaphoreType.DMA,
        pltpu.SemaphoreType.DMA,
    ],
)


def _tc_src4_body(src_ref, out_ref):
    s4 = src_ref[...] * 4
    for c in range(4):
        out_ref[c, :, :] = s4 + c


def _tc_src4(src2):
    rbe = ER // 8
    return pl.pallas_call(
        _tc_src4_body,
        grid=(8,),
        in_specs=[pl.BlockSpec((rbe, EB), lambda i: (i, 0))],
        out_specs=pl.BlockSpec((4, rbe, EB), lambda i: (0, i, 0)),
        out_shape=jax.ShapeDtypeStruct((4, ER, EB), jnp.int32),
    )(src2)


def _tc_pre_body(x_ref, deg_ref, wemb_ref, bemb_ref, wg0_ref,
                 y0_ref, dinv_ref):
    deg = deg_ref[0, :] + deg_ref[1, :] + 1.0
    dinv = (1.0 / jnp.sqrt(deg))[:, None]
    h = jnp.dot(x_ref[...], wemb_ref[...].T, precision=_PREC) + bemb_ref[...]
    y0_ref[...] = jnp.dot(h, wg0_ref[...].T, precision=_PREC) * dinv
    dinv_ref[...] = dinv


def _tc_pre(x, deg2, w_emb, b_emb, w_g0):
    return pl.pallas_call(
        _tc_pre_body,
        grid=(NB,),
        in_specs=[
            pl.BlockSpec((RB, F_IN), lambda i: (i, 0)),
            pl.BlockSpec((NSC, RB), lambda i: (0, i)),
            pl.BlockSpec((H, F_IN), lambda i: (0, 0)),
            pl.BlockSpec((1, H), lambda i: (0, 0)),
            pl.BlockSpec((H, H), lambda i: (0, 0)),
        ],
        out_specs=[
            pl.BlockSpec((RB, H), lambda i: (i, 0)),
            pl.BlockSpec((RB, 1), lambda i: (i, 0)),
        ],
        out_shape=[
            jax.ShapeDtypeStruct((NPAD, H), jnp.float32),
            jax.ShapeDtypeStruct((NPAD, 1), jnp.float32),
        ],
    )(x, deg2, w_emb, b_emb.reshape(1, H), w_g0)


def _tc_mid_body(a_ref, y_ref, dinv_ref, b_ref, w_ref, yout_ref):
    dinv = dinv_ref[...]
    h = jnp.maximum(dinv * (a_ref[...] + y_ref[...]) + b_ref[...], 0.0)
    yout_ref[...] = jnp.dot(h, w_ref[...].T, precision=_PREC) * dinv


def _tc_mid(a, y, dinv, b, w):
    return pl.pallas_call(
        _tc_mid_body,
        grid=(NB,),
        in_specs=[
            pl.BlockSpec((RB, H), lambda i: (i, 0)),
            pl.BlockSpec((RB, H), lambda i: (i, 0)),
            pl.BlockSpec((RB, 1), lambda i: (i, 0)),
            pl.BlockSpec((1, H), lambda i: (0, 0)),
            pl.BlockSpec((H, H), lambda i: (0, 0)),
        ],
        out_specs=pl.BlockSpec((RB, H), lambda i: (i, 0)),
        out_shape=jax.ShapeDtypeStruct((NPAD, H), jnp.float32),
    )(a, y, dinv, b.reshape(1, H), w)


def _tc_final_body(a_ref, y_ref, dinv_ref, bg2_ref, bidx_ref,
                   wih_ref, bih_ref, bhh_ref, win_ref, bin_ref,
                   wout_ref, bout_ref, wc1_ref, bc1_ref, wc2_ref, bc2_ref,
                   out_ref, sums, cnt):
    i = pl.program_id(0)

    @pl.when(i == 0)
    def _():
        sums[...] = jnp.zeros_like(sums)
        cnt[...] = jnp.zeros_like(cnt)

    dinv = dinv_ref[...]
    h3 = jnp.maximum(dinv * (a_ref[...] + y_ref[...]) + bg2_ref[...], 0.0)
    b = bidx_ref[0, 0, :]
    onehot = (lax.broadcasted_iota(jnp.int32, (G, RB), 0)
              == b[None, :]).astype(jnp.float32)
    sums[...] += jnp.dot(onehot, h3, precision=lax.Precision.HIGHEST)
    cnt[...] += jnp.sum(onehot, axis=1, keepdims=True)

    @pl.when(i == NB - 1)
    def _():
        ge = sums[...] / jnp.maximum(cnt[...], 1.0)
        gates = (jnp.dot(ge, wih_ref[...].T, precision=_PREC)
                 + bih_ref[...] + bhh_ref[...])
        i_g = gates[:, :H]
        g_g = gates[:, 2 * H:3 * H]
        o_g = gates[:, 3 * H:]
        c = jax.nn.sigmoid(i_g) * jnp.tanh(g_g)
        hl = jax.nn.sigmoid(o_g) * jnp.tanh(c)
        v = (jnp.dot(hl, win_ref[...][2 * H:3 * H, :].T, precision=_PREC)
             + bin_ref[...][:, 2 * H:3 * H])
        o = jnp.dot(v, wout_ref[...].T, precision=_PREC) + bout_ref[...]
        h1 = jnp.maximum(
            jnp.dot(o, wc1_ref[...].T, precision=_PREC) + bc1_ref[...], 0.0)
        out_ref[...] = jnp.dot(h1, wc2_ref[...].T, precision=_PREC) + bc2_ref[...]


def _tc_final(a, y, dinv, bg2, bidx3, w_ih, b_ih, b_hh, w_in, b_in,
              w_out, b_out, w_c1, b_c1, w_c2, b_c2):
    full = lambda r, c: pl.BlockSpec((r, c), lambda i: (0, 0))
    return pl.pallas_call(
        _tc_final_body,
        grid=(NB,),
        in_specs=[
            pl.BlockSpec((RB, H), lambda i: (i, 0)),
            pl.BlockSpec((RB, H), lambda i: (i, 0)),
            pl.BlockSpec((RB, 1), lambda i: (i, 0)),
            full(1, H),
            pl.BlockSpec((1, 1, RB), lambda i: (i, 0, 0)),
            full(4 * H, H), full(1, 4 * H), full(1, 4 * H),
            full(3 * H, H), full(1, 3 * H),
            full(H, H), full(1, H),
            full(H // 2, H), full(1, H // 2),
            full(NC, H // 2), full(1, NC),
        ],
        out_specs=pl.BlockSpec((G, NC), lambda i: (0, 0)),
        out_shape=jax.ShapeDtypeStruct((G, NC), jnp.float32),
        scratch_shapes=[
            pltpu.VMEM((G, H), jnp.float32),
            pltpu.VMEM((G, 1), jnp.float32),
        ],
    )(a, y, dinv, bg2.reshape(1, H), bidx3,
      w_ih, b_ih.reshape(1, 4 * H), b_hh.reshape(1, 4 * H),
      w_in, b_in.reshape(1, 3 * H),
      w_out, b_out.reshape(1, H),
      w_c1, b_c1.reshape(1, H // 2),
      w_c2, b_c2.reshape(1, NC))


def kernel(x, edge_index, batch_idx, W_emb, b_emb, W_g0, b_g0, W_g1, b_g1,
           W_g2, b_g2, W_ih, W_hh, b_ih, b_hh, W_in, b_in, W_out, b_out,
           W_c1, b_c1, W_c2, b_c2):
    src = edge_index[0]
    dst = edge_index[1]
    pad = EP - E
    srcp = jnp.concatenate([src, jnp.zeros((pad,), src.dtype)])
    dstp = jnp.concatenate([dst, jnp.full((pad,), N, dst.dtype)])
    src2 = srcp.reshape(ER, EB)
    dst2 = dstp.reshape(ER, EB)
    xp = jnp.concatenate([x, jnp.zeros((NPAD - N, F_IN), x.dtype)])
    bidxp = jnp.concatenate(
        [batch_idx, jnp.full((NPAD - N,), G, batch_idx.dtype)])
    bidx3 = bidxp.reshape(NB, 1, RB)

    deg2 = _sc_deg(dst2)
    src4 = _tc_src4(src2)
    y0, dinv = _tc_pre(xp, deg2, W_emb, b_emb, W_g0)
    a0 = _sc_scatter(y0.reshape(4 * NPAD, HC), src4, dst2)
    y1 = _tc_mid(a0, y0, dinv, b_g0, W_g1)
    a1 = _sc_scatter(y1.reshape(4 * NPAD, HC), src4, dst2)
    y2 = _tc_mid(a1, y1, dinv, b_g1, W_g2)
    a2 = _sc_scatter(y2.reshape(4 * NPAD, HC), src4, dst2)
    return _tc_final(a2, y2, dinv, b_g2, bidx3, W_ih, b_ih, b_hh,
                     W_in, b_in, W_out, b_out, W_c1, b_c1, W_c2, b_c2)


# speedup vs baseline: 22.3997x; 1.0725x over previous
import functools

import jax
import jax.numpy as jnp
from jax import lax
from jax.experimental import pallas as pl
from jax.experimental.pallas import tpu as pltpu
from jax.experimental.pallas import tpu_sc as plsc

N = 50000
E = 800000
F_IN = 64
H = 128
G = 128
NC = 2

NSC = 2
NSUB = 16
HC = H // 4

EB = 128
KB = 2
NGRP = 196
LSEG = 7
NBODY = NGRP // (2 * LSEG)
ET = EB * KB * NGRP
EP = ET * NSUB
ER = EP // EB

NP = ET
NPAD = NP
RT = NP // NSUB
ZR = 112

KBD = 7
NGD = 28

RB = 3584
NB = NPAD // RB

_PREC = lax.Precision.DEFAULT
_mesh = functools.partial(
    plsc.VectorSubcoreMesh, core_axis_name="c", subcore_axis_name="s",
    num_cores=NSC, num_subcores=NSUB)
_SC_PARAMS = pltpu.CompilerParams(use_tc_tiling_on_sc=False)


def _sc_deg_body(dst2, deg_out, acc, dstbuf, onesbuf, zbuf):
    cid = lax.axis_index("c")
    sid = lax.axis_index("s")
    zero16 = jnp.zeros((16,), jnp.float32)
    one16 = jnp.ones((16,), jnp.float32)

    def init(i, _):
        zbuf[pl.ds(i * 16, 16)] = zero16
        return 0
    lax.fori_loop(0, (NP // NSUB) // 16, init, 0)

    def initone(i, _):
        onesbuf[pl.ds(i * 16, 16)] = one16
        return 0
    lax.fori_loop(0, EB // 16, initone, 0)

    pltpu.sync_copy(zbuf, acc.at[pl.ds(sid * (NP // NSUB), NP // NSUB)])
    plsc.subcore_barrier()

    def grp(g, _):
        row0 = cid * (ER // NSC) + sid * (KBD * NGD) + g * KBD
        pltpu.sync_copy(dst2.at[pl.ds(row0, KBD)], dstbuf)
        for j in range(KBD):
            pltpu.sync_copy(onesbuf, acc.at[dstbuf.at[j]], add=True)
        return 0
    lax.fori_loop(0, NGD, grp, 0)
    plsc.subcore_barrier()

    pltpu.sync_copy(acc.at[pl.ds(sid * (NP // NSUB), NP // NSUB)],
                    deg_out.at[cid, pl.ds(sid * (NP // NSUB), NP // NSUB)])


_sc_deg = pl.kernel(
    _sc_deg_body,
    out_type=jax.ShapeDtypeStruct((NSC, NP), jnp.float32),
    mesh=_mesh(),
    compiler_params=_SC_PARAMS,
    scratch_types=[
        pltpu.VMEM_SHARED((NP,), jnp.float32),
        pltpu.VMEM((KBD, EB), jnp.int32),
        pltpu.VMEM((EB,), jnp.float32),
        pltpu.VMEM((NP // NSUB,), jnp.float32),
    ],
)


def _sc_scatter_body(y4, src4, dst2, a_out, acc,
                     s4A, dA, s4B, dB, row0b, row1b, row2b, row3b, zbuf,
                     gsem, ssem, isem):
    cid = lax.axis_index("c")
    sid = lax.axis_index("s")
    zero16 = jnp.zeros((16,), jnp.float32)
    SR = LSEG * KB

    def initz(i, _):
        zbuf[i // 2, pl.ds((i % 2) * 16, 16)] = zero16
        return 0
    lax.fori_loop(0, ZR * 2, initz, 0)

    for cc in range(2):
        ck = cid * 2 + cc

        def zc(j, _):
            pltpu.sync_copy(
                zbuf, acc.at[pl.ds(sid * (NP // NSUB) + j * ZR, ZR)])
            return 0
        lax.fori_loop(0, (NP // NSUB) // ZR, zc, 0)
        plsc.subcore_barrier()

        def fire_seg(sg, s4seg, dseg):
            row0 = sid * (NGRP * KB) + sg * SR
            pltpu.async_copy(src4.at[ck, pl.ds(row0, SR)], s4seg, isem)
            pltpu.async_copy(dst2.at[pl.ds(row0, SR)], dseg, isem)

        def wait_seg(s4seg, dseg):
            pltpu.make_async_copy(src4.at[ck, pl.ds(0, SR)], s4seg,
                                  isem).wait()
            pltpu.make_async_copy(dst2.at[pl.ds(0, SR)], dseg, isem).wait()

        def fire_gather(s4seg, lr, rbuf):
            pltpu.async_copy(y4.at[s4seg.at[lr]], rbuf, gsem)

        def wait_gather(s4seg, lr, rbuf):
            pltpu.make_async_copy(y4.at[s4seg.at[lr]], rbuf, gsem).wait()

        def fire_scatter(dseg, lr, rbuf):
            pltpu.async_copy(rbuf, acc.at[dseg.at[lr]], ssem, add=True)

        def wait_scatter(dseg, lr, rbuf):
            pltpu.make_async_copy(rbuf, acc.at[dseg.at[lr]], ssem).wait()

        rbufs = (row0b, row1b, row2b, row3b)

        def locb(lt):
            lt = lt % (2 * SR)
            seg = (s4A, dA) if lt < SR else (s4B, dB)
            return seg[0], seg[1], lt % SR, rbufs[lt % 4]

        def half(lt, first, last):
            s4c, dc, lrc, rc = locb(lt)
            if not first:
                _, dp, lrp, rp = locb(lt - 2)
                wait_scatter(dp, lrp, rp)
            if not last:
                s4n, _, lrn, rn = locb(lt + 2)
                fire_gather(s4n, lrn, rn)
            wait_gather(s4c, lrc, rc)
            fire_scatter(dc, lrc, rc)

        fire_seg(0, s4A, dA)
        wait_seg(s4A, dA)
        fire_gather(s4A, 0, rbufs[0])
        fire_gather(s4A, 1, rbufs[1])

        def body(p, _):
            for lt in range(2):
                @pl.when(p == 0)
                def _():
                    half(lt, True, False)

                @pl.when(p > 0)
                def _():
                    half(lt, False, False)
                if lt == 1:
                    fire_seg(2 * p + 1, s4B, dB)
            for lt in range(2, SR - 2):
                half(lt, False, False)
            wait_seg(s4B, dB)
            for lt in range(SR - 2, SR + 2):
                half(lt, False, False)
                if lt == SR + 1:
                    @pl.when(p < NBODY - 1)
                    def _():
                        fire_seg(2 * p + 2, s4A, dA)
            for lt in range(SR + 2, 2 * SR - 2):
                half(lt, False, False)

            @pl.when(p < NBODY - 1)
            def _():
                wait_seg(s4A, dA)
                half(2 * SR - 2, False, False)
                half(2 * SR - 1, False, False)

            @pl.when(p == NBODY - 1)
            def _():
                half(2 * SR - 2, False, True)
                half(2 * SR - 1, False, True)
            return 0
        lax.fori_loop(0, NBODY, body, 0)
        wait_scatter(dB, SR - 2, rbufs[(2 * SR - 2) % 4])
        wait_scatter(dB, SR - 1, rbufs[(2 * SR - 1) % 4])
        plsc.subcore_barrier()

        pltpu.sync_copy(acc.at[pl.ds(sid * RT, RT)],
                        a_out.at[pl.ds(sid * RT, RT), pl.ds(ck * HC, HC)])
        plsc.subcore_barrier()


_sc_scatter = pl.kernel(
    _sc_scatter_body,
    out_type=jax.ShapeDtypeStruct((NPAD, H), jnp.float32),
    mesh=_mesh(),
    compiler_params=_SC_PARAMS,
    scratch_types=[
        pltpu.VMEM_SHARED((NP, HC), jnp.float32),
        pltpu.VMEM((LSEG * KB, EB), jnp.int32),
        pltpu.VMEM((LSEG * KB, EB), jnp.int32),
        pltpu.VMEM((LSEG * KB, EB), jnp.int32),
        pltpu.VMEM((LSEG * KB, EB), jnp.int32),
        pltpu.VMEM((EB, HC), jnp.float32),
        pltpu.VMEM((EB, HC), jnp.float32),
        pltpu.VMEM((EB, HC), jnp.float32),
        pltpu.VMEM((EB, HC), jnp.float32),
        pltpu.VMEM((ZR, HC), jnp.float32),
        pltpu.SemaphoreType.DMA,
        pltpu.SemaphoreType.DMA,
        pltpu.SemaphoreType.DMA,
    ],
)


def _tc_src4_body(src_ref, out_ref):
    s4 = src_ref[...] * 4
    for c in range(4):
        out_ref[c, :, :] = s4 + c


def _tc_src4(src2):
    rbe = ER // 8
    return pl.pallas_call(
        _tc_src4_body,
        grid=(8,),
        in_specs=[pl.BlockSpec((rbe, EB), lambda i: (i, 0))],
        out_specs=pl.BlockSpec((4, rbe, EB), lambda i: (0, i, 0)),
        out_shape=jax.ShapeDtypeStruct((4, ER, EB), jnp.int32),
    )(src2)


def _tc_pre_body(x_ref, deg_ref, wemb_ref, bemb_ref, wg0_ref,
                 y0_ref, dinv_ref):
    deg = deg_ref[0, :] + deg_ref[1, :] + 1.0
    dinv = (1.0 / jnp.sqrt(deg))[:, None]
    h = jnp.dot(x_ref[...], wemb_ref[...].T, precision=_PREC) + bemb_ref[...]
    y0_ref[...] = jnp.dot(h, wg0_ref[...].T, precision=_PREC) * dinv
    dinv_ref[...] = dinv


def _tc_pre(x, deg2, w_emb, b_emb, w_g0):
    return pl.pallas_call(
        _tc_pre_body,
        grid=(NB,),
        in_specs=[
            pl.BlockSpec((RB, F_IN), lambda i: (i, 0)),
            pl.BlockSpec((NSC, RB), lambda i: (0, i)),
            pl.BlockSpec((H, F_IN), lambda i: (0, 0)),
            pl.BlockSpec((1, H), lambda i: (0, 0)),
            pl.BlockSpec((H, H), lambda i: (0, 0)),
        ],
        out_specs=[
            pl.BlockSpec((RB, H), lambda i: (i, 0)),
            pl.BlockSpec((RB, 1), lambda i: (i, 0)),
        ],
        out_shape=[
            jax.ShapeDtypeStruct((NPAD, H), jnp.float32),
            jax.ShapeDtypeStruct((NPAD, 1), jnp.float32),
        ],
    )(x, deg2, w_emb, b_emb.reshape(1, H), w_g0)


def _tc_mid_body(a_ref, y_ref, dinv_ref, b_ref, w_ref, yout_ref):
    dinv = dinv_ref[...]
    h = jnp.maximum(dinv * (a_ref[...] + y_ref[...]) + b_ref[...], 0.0)
    yout_ref[...] = jnp.dot(h, w_ref[...].T, precision=_PREC) * dinv


def _tc_mid(a, y, dinv, b, w):
    return pl.pallas_call(
        _tc_mid_body,
        grid=(NB,),
        in_specs=[
            pl.BlockSpec((RB, H), lambda i: (i, 0)),
            pl.BlockSpec((RB, H), lambda i: (i, 0)),
            pl.BlockSpec((RB, 1), lambda i: (i, 0)),
            pl.BlockSpec((1, H), lambda i: (0, 0)),
            pl.BlockSpec((H, H), lambda i: (0, 0)),
        ],
        out_specs=pl.BlockSpec((RB, H), lambda i: (i, 0)),
        out_shape=jax.ShapeDtypeStruct((NPAD, H), jnp.float32),
    )(a, y, dinv, b.reshape(1, H), w)


def _tc_final_body(a_ref, y_ref, dinv_ref, bg2_ref, bidx_ref,
                   wih_ref, bih_ref, bhh_ref, win_ref, bin_ref,
                   wout_ref, bout_ref, wc1_ref, bc1_ref, wc2_ref, bc2_ref,
                   out_ref, sums, cnt):
    i = pl.program_id(0)

    @pl.when(i == 0)
    def _():
        sums[...] = jnp.zeros_like(sums)
        cnt[...] = jnp.zeros_like(cnt)

    dinv = dinv_ref[...]
    h3 = jnp.maximum(dinv * (a_ref[...] + y_ref[...]) + bg2_ref[...], 0.0)
    b = bidx_ref[0, 0, :]
    onehot = (lax.broadcasted_iota(jnp.int32, (G, RB), 0)
              == b[None, :]).astype(jnp.float32)
    sums[...] += jnp.dot(onehot, h3, precision=lax.Precision.HIGHEST)
    cnt[...] += jnp.sum(onehot, axis=1, keepdims=True)

    @pl.when(i == NB - 1)
    def _():
        ge = sums[...] / jnp.maximum(cnt[...], 1.0)
        gates = (jnp.dot(ge, wih_ref[...].T, precision=_PREC)
                 + bih_ref[...] + bhh_ref[...])
        i_g = gates[:, :H]
        g_g = gates[:, 2 * H:3 * H]
        o_g = gates[:, 3 * H:]
        c = jax.nn.sigmoid(i_g) * jnp.tanh(g_g)
        hl = jax.nn.sigmoid(o_g) * jnp.tanh(c)
        v = (jnp.dot(hl, win_ref[...][2 * H:3 * H, :].T, precision=_PREC)
             + bin_ref[...][:, 2 * H:3 * H])
        o = jnp.dot(v, wout_ref[...].T, precision=_PREC) + bout_ref[...]
        h1 = jnp.maximum(
            jnp.dot(o, wc1_ref[...].T, precision=_PREC) + bc1_ref[...], 0.0)
        out_ref[...] = jnp.dot(h1, wc2_ref[...].T, precision=_PREC) + bc2_ref[...]


def _tc_final(a, y, dinv, bg2, bidx3, w_ih, b_ih, b_hh, w_in, b_in,
              w_out, b_out, w_c1, b_c1, w_c2, b_c2):
    full = lambda r, c: pl.BlockSpec((r, c), lambda i: (0, 0))
    return pl.pallas_call(
        _tc_final_body,
        grid=(NB,),
        in_specs=[
            pl.BlockSpec((RB, H), lambda i: (i, 0)),
            pl.BlockSpec((RB, H), lambda i: (i, 0)),
            pl.BlockSpec((RB, 1), lambda i: (i, 0)),
            full(1, H),
            pl.BlockSpec((1, 1, RB), lambda i: (i, 0, 0)),
            full(4 * H, H), full(1, 4 * H), full(1, 4 * H),
            full(3 * H, H), full(1, 3 * H),
            full(H, H), full(1, H),
            full(H // 2, H), full(1, H // 2),
            full(NC, H // 2), full(1, NC),
        ],
        out_specs=pl.BlockSpec((G, NC), lambda i: (0, 0)),
        out_shape=jax.ShapeDtypeStruct((G, NC), jnp.float32),
        scratch_shapes=[
            pltpu.VMEM((G, H), jnp.float32),
            pltpu.VMEM((G, 1), jnp.float32),
        ],
    )(a, y, dinv, bg2.reshape(1, H), bidx3,
      w_ih, b_ih.reshape(1, 4 * H), b_hh.reshape(1, 4 * H),
      w_in, b_in.reshape(1, 3 * H),
      w_out, b_out.reshape(1, H),
      w_c1, b_c1.reshape(1, H // 2),
      w_c2, b_c2.reshape(1, NC))


def kernel(x, edge_index, batch_idx, W_emb, b_emb, W_g0, b_g0, W_g1, b_g1,
           W_g2, b_g2, W_ih, W_hh, b_ih, b_hh, W_in, b_in, W_out, b_out,
           W_c1, b_c1, W_c2, b_c2):
    src = edge_index[0]
    dst = edge_index[1]
    pad = EP - E
    srcp = jnp.concatenate([src, jnp.zeros((pad,), src.dtype)])
    dstp = jnp.concatenate([dst, jnp.full((pad,), N, dst.dtype)])
    src2 = srcp.reshape(ER, EB)
    dst2 = dstp.reshape(ER, EB)
    xp = jnp.concatenate([x, jnp.zeros((NPAD - N, F_IN), x.dtype)])
    bidxp = jnp.concatenate(
        [batch_idx, jnp.full((NPAD - N,), G, batch_idx.dtype)])
    bidx3 = bidxp.reshape(NB, 1, RB)

    deg2 = _sc_deg(dst2)
    src4 = _tc_src4(src2)
    y0, dinv = _tc_pre(xp, deg2, W_emb, b_emb, W_g0)
    a0 = _sc_scatter(y0.reshape(4 * NPAD, HC), src4, dst2)
    y1 = _tc_mid(a0, y0, dinv, b_g0, W_g1)
    a1 = _sc_scatter(y1.reshape(4 * NPAD, HC), src4, dst2)
    y2 = _tc_mid(a1, y1, dinv, b_g1, W_g2)
    a2 = _sc_scatter(y2.reshape(4 * NPAD, HC), src4, dst2)
    return _tc_final(a2, y2, dinv, b_g2, bidx3, W_ih, b_ih, b_hh,
                     W_in, b_in, W_out, b_out, W_c1, b_c1, W_c2, b_c2)


# speedup vs baseline: 22.5897x; 1.0085x over previous
import functools

import jax
import jax.numpy as jnp
from jax import lax
from jax.experimental import pallas as pl
from jax.experimental.pallas import tpu as pltpu
from jax.experimental.pallas import tpu_sc as plsc

N = 50000
E = 800000
F_IN = 64
H = 128
G = 128
NC = 2

NSC = 2
NSUB = 16
HC = H // 4

EB = 128
KB = 2
NGRP = 196
LSEG = 7
NBODY = NGRP // (2 * LSEG)
ET = EB * KB * NGRP
EP = ET * NSUB
ER = EP // EB

NP = ET
NPAD = NP
RT = NP // NSUB
ZR = 112

KBD = 7
NGD = 28

RB = 3584
NB = NPAD // RB

_PREC = lax.Precision.DEFAULT
_mesh = functools.partial(
    plsc.VectorSubcoreMesh, core_axis_name="c", subcore_axis_name="s",
    num_cores=NSC, num_subcores=NSUB)
_SC_PARAMS = pltpu.CompilerParams(use_tc_tiling_on_sc=False)


def _sc_deg_body(dst2, deg_out, acc, dstbuf, onesbuf, zbuf):
    cid = lax.axis_index("c")
    sid = lax.axis_index("s")
    zero16 = jnp.zeros((16,), jnp.float32)
    one16 = jnp.ones((16,), jnp.float32)

    def init(i, _):
        zbuf[pl.ds(i * 16, 16)] = zero16
        return 0
    lax.fori_loop(0, (NP // NSUB) // 16, init, 0)

    def initone(i, _):
        onesbuf[pl.ds(i * 16, 16)] = one16
        return 0
    lax.fori_loop(0, EB // 16, initone, 0)

    pltpu.sync_copy(zbuf, acc.at[pl.ds(sid * (NP // NSUB), NP // NSUB)])
    plsc.subcore_barrier()

    def grp(g, _):
        row0 = cid * (ER // NSC) + sid * (KBD * NGD) + g * KBD
        pltpu.sync_copy(dst2.at[pl.ds(row0, KBD)], dstbuf)
        for j in range(KBD):
            pltpu.sync_copy(onesbuf, acc.at[dstbuf.at[j]], add=True)
        return 0
    lax.fori_loop(0, NGD, grp, 0)
    plsc.subcore_barrier()

    pltpu.sync_copy(acc.at[pl.ds(sid * (NP // NSUB), NP // NSUB)],
                    deg_out.at[cid, pl.ds(sid * (NP // NSUB), NP // NSUB)])


_sc_deg = pl.kernel(
    _sc_deg_body,
    out_type=jax.ShapeDtypeStruct((NSC, NP), jnp.float32),
    mesh=_mesh(),
    compiler_params=_SC_PARAMS,
    scratch_types=[
        pltpu.VMEM_SHARED((NP,), jnp.float32),
        pltpu.VMEM((KBD, EB), jnp.int32),
        pltpu.VMEM((EB,), jnp.float32),
        pltpu.VMEM((NP // NSUB,), jnp.float32),
    ],
)


def _sc_scatter_body(y4, src4, dst2, a_out, acc,
                     s4A, dA, s4B, dB, row0b, row1b, row2b, row3b, zbuf,
                     gsem, ssem, isem):
    cid = lax.axis_index("c")
    sid = lax.axis_index("s")
    zero16 = jnp.zeros((16,), jnp.float32)
    SR = LSEG * KB

    def initz(i, _):
        zbuf[i // 2, pl.ds((i % 2) * 16, 16)] = zero16
        return 0
    lax.fori_loop(0, ZR * 2, initz, 0)

    for cc in range(2):
        ck = cid * 2 + cc

        def zc(j, _):
            pltpu.sync_copy(
                zbuf, acc.at[pl.ds(sid * (NP // NSUB) + j * ZR, ZR)])
            return 0
        lax.fori_loop(0, (NP // NSUB) // ZR, zc, 0)
        plsc.subcore_barrier()

        def fire_seg(sg, s4seg, dseg):
            row0 = sid * (NGRP * KB) + sg * SR
            pltpu.async_copy(src4.at[ck, pl.ds(row0, SR)], s4seg, isem)
            pltpu.async_copy(dst2.at[pl.ds(row0, SR)], dseg, isem)

        def wait_seg(s4seg, dseg):
            pltpu.make_async_copy(src4.at[ck, pl.ds(0, SR)], s4seg,
                                  isem).wait()
            pltpu.make_async_copy(dst2.at[pl.ds(0, SR)], dseg, isem).wait()

        def fire_gather(s4seg, lr, rbuf):
            pltpu.async_copy(y4.at[s4seg.at[lr]], rbuf, gsem)

        def wait_gather(s4seg, lr, rbuf):
            pltpu.make_async_copy(y4.at[s4seg.at[lr]], rbuf, gsem).wait()

        def fire_scatter(dseg, lr, rbuf):
            pltpu.async_copy(rbuf, acc.at[dseg.at[lr]], ssem, add=True)

        def wait_scatter(dseg, lr, rbuf):
            pltpu.make_async_copy(rbuf, acc.at[dseg.at[lr]], ssem).wait()

        rbufs = (row0b, row1b, row2b, row3b)

        def locb(lt):
            lt = lt % (2 * SR)
            seg = (s4A, dA) if lt < SR else (s4B, dB)
            return seg[0], seg[1], lt % SR, rbufs[lt % 4]

        def half(lt, first, last):
            s4c, dc, lrc, rc = locb(lt)
            if not first:
                _, dp, lrp, rp = locb(lt - 2)
                wait_scatter(dp, lrp, rp)
            if not last:
                s4n, _, lrn, rn = locb(lt + 2)
                fire_gather(s4n, lrn, rn)
            wait_gather(s4c, lrc, rc)
            fire_scatter(dc, lrc, rc)

        fire_seg(0, s4A, dA)
        wait_seg(s4A, dA)
        fire_gather(s4A, 0, rbufs[0])
        fire_gather(s4A, 1, rbufs[1])

        def body(p, _):
            for lt in range(2):
                @pl.when(p == 0)
                def _():
                    half(lt, True, False)

                @pl.when(p > 0)
                def _():
                    half(lt, False, False)
                if lt == 1:
                    fire_seg(2 * p + 1, s4B, dB)
            for lt in range(2, SR - 2):
                half(lt, False, False)
            wait_seg(s4B, dB)
            for lt in range(SR - 2, SR + 2):
                half(lt, False, False)
                if lt == SR + 1:
                    @pl.when(p < NBODY - 1)
                    def _():
                        fire_seg(2 * p + 2, s4A, dA)
            for lt in range(SR + 2, 2 * SR - 2):
                half(lt, False, False)

            @pl.when(p < NBODY - 1)
            def _():
                wait_seg(s4A, dA)
                half(2 * SR - 2, False, False)
                half(2 * SR - 1, False, False)

            @pl.when(p == NBODY - 1)
            def _():
                half(2 * SR - 2, False, True)
                half(2 * SR - 1, False, True)
            return 0
        lax.fori_loop(0, NBODY, body, 0)
        wait_scatter(dB, SR - 2, rbufs[(2 * SR - 2) % 4])
        wait_scatter(dB, SR - 1, rbufs[(2 * SR - 1) % 4])
        plsc.subcore_barrier()

        pltpu.sync_copy(acc.at[pl.ds(sid * RT, RT)],
                        a_out.at[pl.ds(sid * RT, RT), pl.ds(ck * HC, HC)])
        plsc.subcore_barrier()


_sc_scatter = pl.kernel(
    _sc_scatter_body,
    out_type=jax.ShapeDtypeStruct((NPAD, H), jnp.float32),
    mesh=_mesh(),
    compiler_params=_SC_PARAMS,
    scratch_types=[
        pltpu.VMEM_SHARED((NP, HC), jnp.float32),
        pltpu.VMEM((LSEG * KB, EB), jnp.int32),
        pltpu.VMEM((LSEG * KB, EB), jnp.int32),
        pltpu.VMEM((LSEG * KB, EB), jnp.int32),
        pltpu.VMEM((LSEG * KB, EB), jnp.int32),
        pltpu.VMEM((EB, HC), jnp.float32),
        pltpu.VMEM((EB, HC), jnp.float32),
        pltpu.VMEM((EB, HC), jnp.float32),
        pltpu.VMEM((EB, HC), jnp.float32),
        pltpu.VMEM((ZR, HC), jnp.float32),
        pltpu.SemaphoreType.DMA,
        pltpu.SemaphoreType.DMA,
        pltpu.SemaphoreType.DMA,
    ],
)


def _tc_src4_body(src_ref, out_ref):
    s4 = src_ref[...] * 4
    for c in range(4):
        out_ref[c, :, :] = s4 + c


def _tc_src4(src2):
    rbe = ER // 8
    return pl.pallas_call(
        _tc_src4_body,
        grid=(8,),
        in_specs=[pl.BlockSpec((rbe, EB), lambda i: (i, 0))],
        out_specs=pl.BlockSpec((4, rbe, EB), lambda i: (0, i, 0)),
        out_shape=jax.ShapeDtypeStruct((4, ER, EB), jnp.int32),
    )(src2)


def _tc_pre_body(x_ref, deg_ref, wemb_ref, bemb_ref, wg0_ref,
                 y0_ref, dinv_ref):
    deg = deg_ref[0, :] + deg_ref[1, :] + 1.0
    dinv = (1.0 / jnp.sqrt(deg))[:, None]
    h = jnp.dot(x_ref[...], wemb_ref[...].T, precision=_PREC) + bemb_ref[...]
    y0_ref[...] = jnp.dot(h, wg0_ref[...].T, precision=_PREC) * dinv
    dinv_ref[...] = dinv


def _tc_pre(x, deg2, w_emb, b_emb, w_g0):
    return pl.pallas_call(
        _tc_pre_body,
        grid=(NB,),
        in_specs=[
            pl.BlockSpec((RB, F_IN), lambda i: (i, 0)),
            pl.BlockSpec((NSC, RB), lambda i: (0, i)),
            pl.BlockSpec((H, F_IN), lambda i: (0, 0)),
            pl.BlockSpec((1, H), lambda i: (0, 0)),
            pl.BlockSpec((H, H), lambda i: (0, 0)),
        ],
        out_specs=[
            pl.BlockSpec((RB, H), lambda i: (i, 0)),
            pl.BlockSpec((RB, 1), lambda i: (i, 0)),
        ],
        out_shape=[
            jax.ShapeDtypeStruct((NPAD, H), jnp.float32),
            jax.ShapeDtypeStruct((NPAD, 1), jnp.float32),
        ],
    )(x, deg2, w_emb, b_emb.reshape(1, H), w_g0)


def _tc_mid_body(a_ref, y_ref, dinv_ref, b_ref, w_ref, yout_ref):
    dinv = dinv_ref[...]
    h = jnp.maximum(dinv * (a_ref[...] + y_ref[...]) + b_ref[...], 0.0)
    yout_ref[...] = jnp.dot(h, w_ref[...].T, precision=_PREC) * dinv


def _tc_mid(a, y, dinv, b, w):
    return pl.pallas_call(
        _tc_mid_body,
        grid=(NB,),
        in_specs=[
            pl.BlockSpec((RB, H), lambda i: (i, 0)),
            pl.BlockSpec((RB, H), lambda i: (i, 0)),
            pl.BlockSpec((RB, 1), lambda i: (i, 0)),
            pl.BlockSpec((1, H), lambda i: (0, 0)),
            pl.BlockSpec((H, H), lambda i: (0, 0)),
        ],
        out_specs=pl.BlockSpec((RB, H), lambda i: (i, 0)),
        out_shape=jax.ShapeDtypeStruct((NPAD, H), jnp.float32),
    )(a, y, dinv, b.reshape(1, H), w)


def _tc_final_body(a_ref, y_ref, dinv_ref, bg2_ref, bidx_ref,
                   wih_ref, bih_ref, bhh_ref, win_ref, bin_ref,
                   wout_ref, bout_ref, wc1_ref, bc1_ref, wc2_ref, bc2_ref,
                   out_ref, sums, cnt):
    i = pl.program_id(0)

    @pl.when(i == 0)
    def _():
        sums[...] = jnp.zeros_like(sums)
        cnt[...] = jnp.zeros_like(cnt)

    dinv = dinv_ref[...]
    h3 = jnp.maximum(dinv * (a_ref[...] + y_ref[...]) + bg2_ref[...], 0.0)
    b = bidx_ref[0, 0, :]
    onehot = (lax.broadcasted_iota(jnp.int32, (G, RB), 0)
              == b[None, :]).astype(jnp.float32)
    sums[...] += jnp.dot(onehot, h3, precision=_PREC)
    cnt[...] += jnp.sum(onehot, axis=1, keepdims=True)

    @pl.when(i == NB - 1)
    def _():
        ge = sums[...] / jnp.maximum(cnt[...], 1.0)
        gates = (jnp.dot(ge, wih_ref[...].T, precision=_PREC)
                 + bih_ref[...] + bhh_ref[...])
        i_g = gates[:, :H]
        g_g = gates[:, 2 * H:3 * H]
        o_g = gates[:, 3 * H:]
        c = jax.nn.sigmoid(i_g) * jnp.tanh(g_g)
        hl = jax.nn.sigmoid(o_g) * jnp.tanh(c)
        v = (jnp.dot(hl, win_ref[...][2 * H:3 * H, :].T, precision=_PREC)
             + bin_ref[...][:, 2 * H:3 * H])
        o = jnp.dot(v, wout_ref[...].T, precision=_PREC) + bout_ref[...]
        h1 = jnp.maximum(
            jnp.dot(o, wc1_ref[...].T, precision=_PREC) + bc1_ref[...], 0.0)
        out_ref[...] = jnp.dot(h1, wc2_ref[...].T, precision=_PREC) + bc2_ref[...]


def _tc_final(a, y, dinv, bg2, bidx3, w_ih, b_ih, b_hh, w_in, b_in,
              w_out, b_out, w_c1, b_c1, w_c2, b_c2):
    full = lambda r, c: pl.BlockSpec((r, c), lambda i: (0, 0))
    return pl.pallas_call(
        _tc_final_body,
        grid=(NB,),
        in_specs=[
            pl.BlockSpec((RB, H), lambda i: (i, 0)),
            pl.BlockSpec((RB, H), lambda i: (i, 0)),
            pl.BlockSpec((RB, 1), lambda i: (i, 0)),
            full(1, H),
            pl.BlockSpec((1, 1, RB), lambda i: (i, 0, 0)),
            full(4 * H, H), full(1, 4 * H), full(1, 4 * H),
            full(3 * H, H), full(1, 3 * H),
            full(H, H), full(1, H),
            full(H // 2, H), full(1, H // 2),
            full(NC, H // 2), full(1, NC),
        ],
        out_specs=pl.BlockSpec((G, NC), lambda i: (0, 0)),
        out_shape=jax.ShapeDtypeStruct((G, NC), jnp.float32),
        scratch_shapes=[
            pltpu.VMEM((G, H), jnp.float32),
            pltpu.VMEM((G, 1), jnp.float32),
        ],
    )(a, y, dinv, bg2.reshape(1, H), bidx3,
      w_ih, b_ih.reshape(1, 4 * H), b_hh.reshape(1, 4 * H),
      w_in, b_in.reshape(1, 3 * H),
      w_out, b_out.reshape(1, H),
      w_c1, b_c1.reshape(1, H // 2),
      w_c2, b_c2.reshape(1, NC))


def kernel(x, edge_index, batch_idx, W_emb, b_emb, W_g0, b_g0, W_g1, b_g1,
           W_g2, b_g2, W_ih, W_hh, b_ih, b_hh, W_in, b_in, W_out, b_out,
           W_c1, b_c1, W_c2, b_c2):
    src = edge_index[0]
    dst = edge_index[1]
    pad = EP - E
    srcp = jnp.concatenate([src, jnp.zeros((pad,), src.dtype)])
    dstp = jnp.concatenate([dst, jnp.full((pad,), N, dst.dtype)])
    src2 = srcp.reshape(ER, EB)
    dst2 = dstp.reshape(ER, EB)
    xp = jnp.concatenate([x, jnp.zeros((NPAD - N, F_IN), x.dtype)])
    bidxp = jnp.concatenate(
        [batch_idx, jnp.full((NPAD - N,), G, batch_idx.dtype)])
    bidx3 = bidxp.reshape(NB, 1, RB)

    deg2 = _sc_deg(dst2)
    src4 = _tc_src4(src2)
    y0, dinv = _tc_pre(xp, deg2, W_emb, b_emb, W_g0)
    a0 = _sc_scatter(y0.reshape(4 * NPAD, HC), src4, dst2)
    y1 = _tc_mid(a0, y0, dinv, b_g0, W_g1)
    a1 = _sc_scatter(y1.reshape(4 * NPAD, HC), src4, dst2)
    y2 = _tc_mid(a1, y1, dinv, b_g1, W_g2)
    a2 = _sc_scatter(y2.reshape(4 * NPAD, HC), src4, dst2)
    return _tc_final(a2, y2, dinv, b_g2, bidx3, W_ih, b_ih, b_hh,
                     W_in, b_in, W_out, b_out, W_c1, b_c1, W_c2, b_c2)
